# Initial kernel scaffold; baseline (speedup 1.0000x reference)
import jax
import jax.numpy as jnp
from jax.experimental import pallas as pl


def kernel(x, conv_w, conv_b, fc1_w, fc1_b, fc2_w, fc2_b):
    raise NotImplementedError("write your pallas kernel here")


# speedup vs baseline: 3.5686x; 3.5686x over previous
import jax
import jax.numpy as jnp
from jax.experimental import pallas as pl
from jax.experimental.pallas import tpu as pltpu


_NB = 128


def _fused_body(xw_ref, wc_ref, bc_ref, w1_ref, b1_ref, w2_ref, b2_ref, o_ref):
    nb = o_ref.shape[0]
    lhs = xw_ref[...].reshape(12 * nb, 192)
    acc = jnp.dot(lhs, wc_ref[...], preferred_element_type=jnp.float32)
    z = jnp.maximum(jnp.maximum(acc[:, 0:384], acc[:, 384:768]),
                    jnp.maximum(acc[:, 768:1152], acc[:, 1152:1536]))
    z = jnp.maximum(z + bc_ref[...], 0.0).astype(jnp.bfloat16)
    z3 = z.reshape(12, nb, 384)
    h = jnp.dot(z3[0], w1_ref[0], preferred_element_type=jnp.float32)
    for p in range(1, 12):
        h = h + jnp.dot(z3[p], w1_ref[p], preferred_element_type=jnp.float32)
    h = jnp.maximum(h + b1_ref[...], 0.0)
    o_ref[...] = jnp.dot(h, w2_ref[...],
                         preferred_element_type=jnp.float32) + b2_ref[...]


def _band_weights(conv_w):
    s = jnp.arange(6).reshape(6, 1, 1, 1, 1)
    j = jnp.arange(32).reshape(1, 32, 1, 1, 1)
    dy = jnp.arange(2).reshape(1, 1, 2, 1, 1)
    dx = jnp.arange(2).reshape(1, 1, 1, 2, 1)
    pw = jnp.arange(12).reshape(1, 1, 1, 1, 12)
    ky = s - dy
    kx = j - 2 * pw - dx
    valid = (ky >= 0) & (ky <= 4) & (kx >= 0) & (kx <= 4)
    wt = conv_w[:, 0].transpose(1, 2, 0)
    vals = wt[jnp.clip(ky, 0, 4), jnp.clip(kx, 0, 4)]
    wc = jnp.where(valid[..., None], vals, 0.0)
    return wc.reshape(192, 1536).astype(jnp.bfloat16)


def kernel(x, conv_w, conv_b, fc1_w, fc1_b, fc2_w, fc2_b):
    n = x.shape[0]
    nb = _NB if n >= _NB else 8
    n_pad = -(-n // nb) * nb

    xp = jnp.pad(x[:, 0], ((0, n_pad - n), (0, 0), (0, 4)))
    xb = xp.astype(jnp.bfloat16)
    xw = jnp.stack(
        [xb[:, 2 * p:2 * p + 6, :].reshape(n_pad, 192) for p in range(12)],
        axis=0)

    wc = _band_weights(conv_w)
    bc = jnp.tile(conv_b, 12).reshape(1, 384)
    w1r = (fc1_w.reshape(128, 32, 12, 12)
           .transpose(2, 3, 1, 0)
           .reshape(12, 384, 128)
           .astype(jnp.bfloat16))
    b1 = fc1_b.reshape(1, 128)
    w2p = jnp.zeros((128, 128), jnp.float32).at[:, :10].set(fc2_w.T)
    b2p = jnp.zeros((1, 128), jnp.float32).at[0, :10].set(fc2_b)

    out = pl.pallas_call(
        _fused_body,
        out_shape=jax.ShapeDtypeStruct((n_pad, 128), jnp.float32),
        grid=(n_pad // nb,),
        in_specs=[
            pl.BlockSpec((12, nb, 192), lambda i: (0, i, 0)),
            pl.BlockSpec((192, 1536), lambda i: (0, 0)),
            pl.BlockSpec((1, 384), lambda i: (0, 0)),
            pl.BlockSpec((12, 384, 128), lambda i: (0, 0, 0)),
            pl.BlockSpec((1, 128), lambda i: (0, 0)),
            pl.BlockSpec((128, 128), lambda i: (0, 0)),
            pl.BlockSpec((1, 128), lambda i: (0, 0)),
        ],
        out_specs=pl.BlockSpec((nb, 128), lambda i: (i, 0)),
        compiler_params=pltpu.CompilerParams(
            dimension_semantics=("parallel",)),
        name="fused_convnet",
    )(xw, wc, bc, w1r, b1, w2p, b2p)
    return out[:n, :10]


# speedup vs baseline: 4.6645x; 1.3071x over previous
import jax
import jax.numpy as jnp
from jax.experimental import pallas as pl
from jax.experimental.pallas import tpu as pltpu


_NB = 128


def _fused_body(xw_ref, wc_ref, bc_ref, w1_ref, b1_ref, w2_ref, b2_ref, o_ref):
    nb = o_ref.shape[0]
    xr = xw_ref[...]
    lhs = jnp.concatenate([xr[0:12], xr[1:13], xr[2:14]],
                          axis=2).reshape(12 * nb, 192)
    acc = jnp.dot(lhs, wc_ref[...], preferred_element_type=jnp.float32)
    z = jnp.maximum(jnp.maximum(acc[:, 0:384], acc[:, 384:768]),
                    jnp.maximum(acc[:, 768:1152], acc[:, 1152:1536]))
    z = jnp.maximum(z + bc_ref[...], 0.0).astype(jnp.bfloat16)
    z3 = z.reshape(12, nb, 384)
    h = jnp.dot(z3[0], w1_ref[0], preferred_element_type=jnp.float32)
    for p in range(1, 12):
        h = h + jnp.dot(z3[p], w1_ref[p], preferred_element_type=jnp.float32)
    h = jnp.maximum(h + b1_ref[...], 0.0)
    o_ref[...] = jnp.dot(h, w2_ref[...],
                         preferred_element_type=jnp.float32) + b2_ref[...]


def _band_weights(conv_w):
    s = jnp.arange(6).reshape(6, 1, 1, 1, 1)
    j = jnp.arange(32).reshape(1, 32, 1, 1, 1)
    dy = jnp.arange(2).reshape(1, 1, 2, 1, 1)
    dx = jnp.arange(2).reshape(1, 1, 1, 2, 1)
    pw = jnp.arange(12).reshape(1, 1, 1, 1, 12)
    ky = s - dy
    kx = j - 2 * pw - dx
    valid = (ky >= 0) & (ky <= 4) & (kx >= 0) & (kx <= 4)
    wt = conv_w[:, 0].transpose(1, 2, 0)
    vals = wt[jnp.clip(ky, 0, 4), jnp.clip(kx, 0, 4)]
    wc = jnp.where(valid[..., None], vals, 0.0)
    return wc.reshape(192, 1536).astype(jnp.bfloat16)


def kernel(x, conv_w, conv_b, fc1_w, fc1_b, fc2_w, fc2_b):
    n = x.shape[0]
    nb = _NB if n >= _NB else 8
    n_pad = -(-n // nb) * nb

    xp = jnp.pad(x[:, 0], ((0, n_pad - n), (0, 0), (0, 4)))
    xw = xp.astype(jnp.bfloat16).reshape(n_pad, 14, 64).transpose(1, 0, 2)

    wc = _band_weights(conv_w)
    bc = jnp.tile(conv_b, 12).reshape(1, 384)
    w1r = (fc1_w.reshape(128, 32, 12, 12)
           .transpose(2, 3, 1, 0)
           .reshape(12, 384, 128)
           .astype(jnp.bfloat16))
    b1 = fc1_b.reshape(1, 128)
    w2p = jnp.zeros((128, 128), jnp.float32).at[:, :10].set(fc2_w.T)
    b2p = jnp.zeros((1, 128), jnp.float32).at[0, :10].set(fc2_b)

    out = pl.pallas_call(
        _fused_body,
        out_shape=jax.ShapeDtypeStruct((n_pad, 128), jnp.float32),
        grid=(n_pad // nb,),
        in_specs=[
            pl.BlockSpec((14, nb, 64), lambda i: (0, i, 0)),
            pl.BlockSpec((192, 1536), lambda i: (0, 0)),
            pl.BlockSpec((1, 384), lambda i: (0, 0)),
            pl.BlockSpec((12, 384, 128), lambda i: (0, 0, 0)),
            pl.BlockSpec((1, 128), lambda i: (0, 0)),
            pl.BlockSpec((128, 128), lambda i: (0, 0)),
            pl.BlockSpec((1, 128), lambda i: (0, 0)),
        ],
        out_specs=pl.BlockSpec((nb, 128), lambda i: (i, 0)),
        compiler_params=pltpu.CompilerParams(
            dimension_semantics=("parallel",)),
        name="fused_convnet",
    )(xw, wc, bc, w1r, b1, w2p, b2p)
    return out[:n, :10]
